# Initial kernel scaffold; baseline (speedup 1.0000x reference)
import jax
import jax.numpy as jnp
from jax import lax
from jax.experimental import pallas as pl
from jax.experimental.pallas import tpu as pltpu, tpu_sc as plsc

N_USERS = 5000
N_NODES = 10000
E = 320000
D = 128
B = 4096

NC, NS, L = 2, 16, 16
EPT = E // NS
NCHUNK = EPT // L
RPT = N_NODES // NS
ZR = 25
BPT = B // NS

_f32 = jnp.float32
_i32 = jnp.int32


def _layer_tile(ego, srcr, dstr, valr, outr, idx_s, idx_d, vals, rows, zbuf,
                acc, sem, s):
    base = s * EPT
    pltpu.sync_copy(srcr.at[pl.ds(base, EPT)], idx_s)
    pltpu.sync_copy(dstr.at[pl.ds(base, EPT)], idx_d)
    pltpu.sync_copy(valr.at[pl.ds(base, EPT)], vals)

    rbase = s * RPT

    @pl.loop(0, RPT // ZR)
    def _zero(k):
        pltpu.sync_copy(zbuf, acc.at[pl.ds(rbase + k * ZR, ZR)])

    plsc.subcore_barrier()

    lanes = lax.iota(_i32, L)

    @pl.loop(0, NCHUNK)
    def _edges(ch):
        off = ch * L
        sv = idx_s[pl.ds(off, L)]
        vv = vals[pl.ds(off, L)]
        pltpu.async_copy(ego.at[sv], rows, sem).wait()

        @pl.loop(0, D)
        def _scale(j):
            ji = jnp.full((L,), j, _i32)
            colv = plsc.load_gather(rows, [lanes, ji])
            plsc.store_scatter(rows, [lanes, ji], colv * vv)

        dv = idx_d[pl.ds(off, L)]
        pltpu.sync_copy(rows, acc.at[dv], add=True)

    plsc.subcore_barrier()
    pltpu.sync_copy(acc.at[pl.ds(rbase, RPT)], outr.at[pl.ds(rbase, RPT)])


def _layer_body(ego_a, ego_b, src_a, dst_a, val_a, src_b, dst_b, val_b,
                out_a, out_b, idx_s, idx_d, vals, rows, zbuf, acc, sem):
    c = lax.axis_index("c")
    s = lax.axis_index("s")

    @pl.loop(0, ZR)
    def _fill(r):
        for j in range(D // L):
            zbuf[r, pl.ds(j * L, L)] = jnp.zeros((L,), _f32)

    @pl.when(c == 0)
    def _():
        _layer_tile(ego_a, src_a, dst_a, val_a, out_a,
                    idx_s, idx_d, vals, rows, zbuf, acc, sem, s)

    @pl.when(c == 1)
    def _():
        _layer_tile(ego_b, src_b, dst_b, val_b, out_b,
                    idx_s, idx_d, vals, rows, zbuf, acc, sem, s)


def _score_tile(e0, e1, e2, e3, uref, iref, outr, uix, iix,
                ub0, ub1, ub2, ub3, ib0, ib1, ib2, ib3, obuf, sem, s):
    base = s * BPT
    pltpu.sync_copy(uref.at[pl.ds(base, BPT)], uix)
    pltpu.sync_copy(iref.at[pl.ds(base, BPT)], iix)

    @pl.loop(0, BPT // L)
    def _pairs(ch):
        off = ch * L
        uv = uix[pl.ds(off, L)]
        iv = iix[pl.ds(off, L)] + N_USERS
        cps = [
            pltpu.async_copy(e0.at[uv], ub0, sem),
            pltpu.async_copy(e1.at[uv], ub1, sem),
            pltpu.async_copy(e2.at[uv], ub2, sem),
            pltpu.async_copy(e3.at[uv], ub3, sem),
            pltpu.async_copy(e0.at[iv], ib0, sem),
            pltpu.async_copy(e1.at[iv], ib1, sem),
            pltpu.async_copy(e2.at[iv], ib2, sem),
            pltpu.async_copy(e3.at[iv], ib3, sem),
        ]
        for cp in cps:
            cp.wait()

        lanes = lax.iota(_i32, L)

        @pl.loop(0, D, init_carry=jnp.zeros((L,), _f32))
        def _dot(j, acc):
            ji = jnp.full((L,), j, _i32)
            uc = (plsc.load_gather(ub0, [lanes, ji])
                  + plsc.load_gather(ub1, [lanes, ji])
                  + plsc.load_gather(ub2, [lanes, ji])
                  + plsc.load_gather(ub3, [lanes, ji]))
            ic = (plsc.load_gather(ib0, [lanes, ji])
                  + plsc.load_gather(ib1, [lanes, ji])
                  + plsc.load_gather(ib2, [lanes, ji])
                  + plsc.load_gather(ib3, [lanes, ji]))
            return acc + uc * ic

        obuf[pl.ds(off, L)] = _dot * (1.0 / 16.0)

    pltpu.sync_copy(obuf, outr.at[pl.ds(base, BPT)])


def _score_body(a0, a1, a2, a3, b0, b1, b2, b3, uaref, iaref, ubref, ibref,
                out_a, out_b, uix, iix,
                ub0, ub1, ub2, ub3, ib0, ib1, ib2, ib3, obuf, sem):
    c = lax.axis_index("c")
    s = lax.axis_index("s")

    @pl.when(c == 0)
    def _():
        _score_tile(a0, a1, a2, a3, uaref, iaref, out_a, uix, iix,
                    ub0, ub1, ub2, ub3, ib0, ib1, ib2, ib3, obuf, sem, s)

    @pl.when(c == 1)
    def _():
        _score_tile(b0, b1, b2, b3, ubref, ibref, out_b, uix, iix,
                    ub0, ub1, ub2, ub3, ib0, ib1, ib2, ib3, obuf, sem, s)


_MESH = plsc.VectorSubcoreMesh(core_axis_name="c", subcore_axis_name="s")

_layer_call = pl.kernel(
    _layer_body,
    out_type=[jax.ShapeDtypeStruct((N_NODES, D), _f32)] * 2,
    mesh=_MESH,
    scratch_types=[
        pltpu.VMEM((EPT,), _i32),
        pltpu.VMEM((EPT,), _i32),
        pltpu.VMEM((EPT,), _f32),
        pltpu.VMEM((L, D), _f32),
        pltpu.VMEM((ZR, D), _f32),
        pltpu.VMEM_SHARED((N_NODES, D), _f32),
        pltpu.SemaphoreType.DMA,
    ],
)

_score_call = pl.kernel(
    _score_body,
    out_type=[jax.ShapeDtypeStruct((B,), _f32)] * 2,
    mesh=_MESH,
    scratch_types=[
        pltpu.VMEM((BPT,), _i32),
        pltpu.VMEM((BPT,), _i32),
        pltpu.VMEM((L, D), _f32),
        pltpu.VMEM((L, D), _f32),
        pltpu.VMEM((L, D), _f32),
        pltpu.VMEM((L, D), _f32),
        pltpu.VMEM((L, D), _f32),
        pltpu.VMEM((L, D), _f32),
        pltpu.VMEM((L, D), _f32),
        pltpu.VMEM((L, D), _f32),
        pltpu.VMEM((BPT,), _f32),
        pltpu.SemaphoreType.DMA,
    ],
)


def kernel(uA, iA, uB, iB, adj_a_idx, adj_a_val, adj_b_idx, adj_b_val,
           ua_idx, ia_idx, ub_idx, ib_idx):
    ego_a = jnp.concatenate([uA, iA], axis=0)
    ego_b = jnp.concatenate([uB, iB], axis=0)
    src_a, dst_a = adj_a_idx[1], adj_a_idx[0]
    src_b, dst_b = adj_b_idx[1], adj_b_idx[0]

    a1, b1 = _layer_call(ego_a, ego_b, src_a, dst_a, adj_a_val,
                         src_b, dst_b, adj_b_val)
    a2, b2 = _layer_call(a1, b1, src_a, dst_a, adj_a_val,
                         src_b, dst_b, adj_b_val)
    a3, b3 = _layer_call(a2, b2, src_a, dst_a, adj_a_val,
                         src_b, dst_b, adj_b_val)

    sa, sb = _score_call(ego_a, a1, a2, a3, ego_b, b1, b2, b3,
                         ua_idx, ia_idx, ub_idx, ib_idx)
    return (sa, sb)


# speedup vs baseline: 1.8312x; 1.8312x over previous
import jax
import jax.numpy as jnp
from jax import lax
from jax.experimental import pallas as pl
from jax.experimental.pallas import tpu as pltpu, tpu_sc as plsc

N_USERS = 5000
N_NODES = 10000
E = 320000
D = 128
B = 4096

NC, NS, L = 2, 16, 16
EPT = E // NS
EBLK = 2000
NCHUNK = EBLK // L
RPT = 624
ZR = 16
BPT = B // NS

_f32 = jnp.float32
_i32 = jnp.int32


def _layer_tile(ego, srcr, dstr, valr, outr, idx_s, idx_d, vals, rows, zbuf,
                acc, sem, s):
    base = s * EPT
    rbase = s * RPT
    tail = NS * RPT

    @pl.loop(0, RPT // ZR)
    def _zero(k):
        pltpu.sync_copy(zbuf, acc.at[pl.ds(rbase + k * ZR, ZR)])

    @pl.when(s == NS - 1)
    def _zero_tail():
        pltpu.sync_copy(zbuf, acc.at[pl.ds(tail, ZR)])

    plsc.subcore_barrier()

    @pl.loop(0, EPT // EBLK)
    def _blk(b):
        ebase = base + b * EBLK
        pltpu.sync_copy(srcr.at[pl.ds(ebase, EBLK)], idx_s)
        pltpu.sync_copy(dstr.at[pl.ds(ebase, EBLK)], idx_d)
        pltpu.sync_copy(valr.at[pl.ds(ebase, EBLK)], vals)

        @pl.loop(0, NCHUNK)
        def _edges(ch):
            off = ch * L
            sv = idx_s[pl.ds(off, L)]
            pltpu.async_copy(ego.at[sv], rows, sem).wait()

            vv = vals[pl.ds(off, L)]
            for e in range(L):
                splat = jnp.full((L,), vv[e], _f32)
                for j in range(D // L):
                    sl = pl.ds(j * L, L)
                    rows[e, sl] = rows[e, sl] * splat

            dv = idx_d[pl.ds(off, L)]
            pltpu.sync_copy(rows, acc.at[dv], add=True)

    plsc.subcore_barrier()
    pltpu.sync_copy(acc.at[pl.ds(rbase, RPT)], outr.at[pl.ds(rbase, RPT)])

    @pl.when(s == NS - 1)
    def _wb_tail():
        pltpu.sync_copy(acc.at[pl.ds(tail, ZR)], outr.at[pl.ds(tail, ZR)])


def _layer_body(ego_a, ego_b, src_a, dst_a, val_a, src_b, dst_b, val_b,
                out_a, out_b, idx_s, idx_d, vals, rows, zbuf, acc, sem):
    c = lax.axis_index("c")
    s = lax.axis_index("s")

    @pl.loop(0, ZR)
    def _fill(r):
        for j in range(D // L):
            zbuf[r, pl.ds(j * L, L)] = jnp.zeros((L,), _f32)

    @pl.when(c == 0)
    def _():
        _layer_tile(ego_a, src_a, dst_a, val_a, out_a,
                    idx_s, idx_d, vals, rows, zbuf, acc, sem, s)

    @pl.when(c == 1)
    def _():
        _layer_tile(ego_b, src_b, dst_b, val_b, out_b,
                    idx_s, idx_d, vals, rows, zbuf, acc, sem, s)


def _score_tile(e0, e1, e2, e3, uref, iref, outr, uix, iix,
                ub0, ub1, ub2, ub3, ib0, ib1, ib2, ib3, obuf, sem, s):
    base = s * BPT
    pltpu.sync_copy(uref.at[pl.ds(base, BPT)], uix)
    pltpu.sync_copy(iref.at[pl.ds(base, BPT)], iix)

    @pl.loop(0, BPT // L)
    def _pairs(ch):
        off = ch * L
        uv = uix[pl.ds(off, L)]
        iv = iix[pl.ds(off, L)] + N_USERS
        cps = [
            pltpu.async_copy(e0.at[uv], ub0, sem),
            pltpu.async_copy(e1.at[uv], ub1, sem),
            pltpu.async_copy(e2.at[uv], ub2, sem),
            pltpu.async_copy(e3.at[uv], ub3, sem),
            pltpu.async_copy(e0.at[iv], ib0, sem),
            pltpu.async_copy(e1.at[iv], ib1, sem),
            pltpu.async_copy(e2.at[iv], ib2, sem),
            pltpu.async_copy(e3.at[iv], ib3, sem),
        ]
        for cp in cps:
            cp.wait()

        lanes = lax.iota(_i32, L)
        svec = jnp.zeros((L,), _f32)
        for e in range(L):
            acc = jnp.zeros((L,), _f32)
            for j in range(D // L):
                sl = pl.ds(j * L, L)
                us = ub0[e, sl] + ub1[e, sl] + ub2[e, sl] + ub3[e, sl]
                vs = ib0[e, sl] + ib1[e, sl] + ib2[e, sl] + ib3[e, sl]
                acc = acc + us * vs
            for m in (8, 4, 2, 1):
                acc = acc + acc.at[lanes ^ m].get(mode="promise_in_bounds")
            svec = jnp.where(lanes == e, acc, svec)
        obuf[pl.ds(off, L)] = svec * (1.0 / 16.0)

    pltpu.sync_copy(obuf, outr.at[pl.ds(base, BPT)])


def _score_body(a0, a1, a2, a3, b0, b1, b2, b3, uaref, iaref, ubref, ibref,
                out_a, out_b, uix, iix,
                ub0, ub1, ub2, ub3, ib0, ib1, ib2, ib3, obuf, sem):
    c = lax.axis_index("c")
    s = lax.axis_index("s")

    @pl.when(c == 0)
    def _():
        _score_tile(a0, a1, a2, a3, uaref, iaref, out_a, uix, iix,
                    ub0, ub1, ub2, ub3, ib0, ib1, ib2, ib3, obuf, sem, s)

    @pl.when(c == 1)
    def _():
        _score_tile(b0, b1, b2, b3, ubref, ibref, out_b, uix, iix,
                    ub0, ub1, ub2, ub3, ib0, ib1, ib2, ib3, obuf, sem, s)


_MESH = plsc.VectorSubcoreMesh(core_axis_name="c", subcore_axis_name="s")

_layer_call = pl.kernel(
    _layer_body,
    out_type=[jax.ShapeDtypeStruct((N_NODES, D), _f32)] * 2,
    mesh=_MESH,
    scratch_types=[
        pltpu.VMEM((EBLK,), _i32),
        pltpu.VMEM((EBLK,), _i32),
        pltpu.VMEM((EBLK,), _f32),
        pltpu.VMEM((L, D), _f32),
        pltpu.VMEM((ZR, D), _f32),
        pltpu.VMEM_SHARED((N_NODES, D), _f32),
        pltpu.SemaphoreType.DMA,
    ],
)

_score_call = pl.kernel(
    _score_body,
    out_type=[jax.ShapeDtypeStruct((B,), _f32)] * 2,
    mesh=_MESH,
    scratch_types=[
        pltpu.VMEM((BPT,), _i32),
        pltpu.VMEM((BPT,), _i32),
        pltpu.VMEM((L, D), _f32),
        pltpu.VMEM((L, D), _f32),
        pltpu.VMEM((L, D), _f32),
        pltpu.VMEM((L, D), _f32),
        pltpu.VMEM((L, D), _f32),
        pltpu.VMEM((L, D), _f32),
        pltpu.VMEM((L, D), _f32),
        pltpu.VMEM((L, D), _f32),
        pltpu.VMEM((BPT,), _f32),
        pltpu.SemaphoreType.DMA,
    ],
)


def kernel(uA, iA, uB, iB, adj_a_idx, adj_a_val, adj_b_idx, adj_b_val,
           ua_idx, ia_idx, ub_idx, ib_idx):
    ego_a = jnp.concatenate([uA, iA], axis=0)
    ego_b = jnp.concatenate([uB, iB], axis=0)
    src_a, dst_a = adj_a_idx[1], adj_a_idx[0]
    src_b, dst_b = adj_b_idx[1], adj_b_idx[0]

    a1, b1 = _layer_call(ego_a, ego_b, src_a, dst_a, adj_a_val,
                         src_b, dst_b, adj_b_val)
    a2, b2 = _layer_call(a1, b1, src_a, dst_a, adj_a_val,
                         src_b, dst_b, adj_b_val)
    a3, b3 = _layer_call(a2, b2, src_a, dst_a, adj_a_val,
                         src_b, dst_b, adj_b_val)

    sa, sb = _score_call(ego_a, a1, a2, a3, ego_b, b1, b2, b3,
                         ua_idx, ia_idx, ub_idx, ib_idx)
    return (sa, sb)


# speedup vs baseline: 5.0535x; 2.7597x over previous
import jax
import jax.numpy as jnp
from jax import lax
from jax.experimental import pallas as pl
from jax.experimental.pallas import tpu as pltpu, tpu_sc as plsc

N_USERS = 5000
N_NODES = 10000
E = 320000
D = 128
B = 4096

NC, NS, L = 2, 16, 16
EPT = E // NS
EBLK = 800
K = 32
SUB = EBLK // K
NBO = EPT // EBLK
RPT = 624
ZR = 16
BPT = B // NS

_f32 = jnp.float32
_i32 = jnp.int32


def _scale_rows(gbuf, mbuf, vals, off):
    @pl.loop(0, K // L)
    def _grp(g):
        vv = vals[pl.ds(off + g * L, L)]
        for e in range(L):
            splat = jnp.full((L,), vv[e], _f32)
            r = g * L + e
            for j in range(D // L):
                sl = pl.ds(j * L, L)
                mbuf[r, sl] = gbuf[r, sl] * splat


def _sub_block(ego, acc, idx_s, idx_d, vals, gbuf, mbuf, dbuf, gsem, ssem, sb):
    off = sb * K
    pltpu.make_async_copy(ego.at[pl.ds(0, K)], gbuf, gsem).wait()

    @pl.when(sb >= 2)
    def _():
        pltpu.make_async_copy(mbuf, acc.at[pl.ds(0, K)], ssem).wait()

    _scale_rows(gbuf, mbuf, vals, off)

    @pl.when(sb + 2 < SUB)
    def _():
        pltpu.async_copy(ego.at[idx_s.at[pl.ds((sb + 2) * K, K)]], gbuf, gsem)

    @pl.loop(0, K // L)
    def _dst(g):
        dbuf[pl.ds(g * L, L)] = idx_d[pl.ds(off + g * L, L)]
    pltpu.async_copy(mbuf, acc.at[dbuf], ssem, add=True)


def _layer_tile(ego, srcr, dstr, valr, outr, idx_s, idx_d, vals,
                gbuf0, gbuf1, mbuf0, mbuf1, dbuf0, dbuf1, zbuf,
                acc, gsem0, gsem1, ssem0, ssem1, s):
    base = s * EPT
    rbase = s * RPT
    tail = NS * RPT

    @pl.loop(0, RPT // ZR)
    def _zero(k):
        pltpu.sync_copy(zbuf, acc.at[pl.ds(rbase + k * ZR, ZR)])

    @pl.when(s == NS - 1)
    def _zero_tail():
        pltpu.sync_copy(zbuf, acc.at[pl.ds(tail, ZR)])

    plsc.subcore_barrier()

    @pl.loop(0, NBO)
    def _blk(b):
        ebase = base + b * EBLK
        pltpu.sync_copy(srcr.at[pl.ds(ebase, EBLK)], idx_s)
        pltpu.sync_copy(dstr.at[pl.ds(ebase, EBLK)], idx_d)
        pltpu.sync_copy(valr.at[pl.ds(ebase, EBLK)], vals)

        pltpu.async_copy(ego.at[idx_s.at[pl.ds(0, K)]], gbuf0, gsem0)
        pltpu.async_copy(ego.at[idx_s.at[pl.ds(K, K)]], gbuf1, gsem1)

        @pl.loop(0, SUB)
        def _sub(sb):
            @pl.when(sb % 2 == 0)
            def _():
                _sub_block(ego, acc, idx_s, idx_d, vals,
                           gbuf0, mbuf0, dbuf0, gsem0, ssem0, sb)

            @pl.when(sb % 2 == 1)
            def _():
                _sub_block(ego, acc, idx_s, idx_d, vals,
                           gbuf1, mbuf1, dbuf1, gsem1, ssem1, sb)

        pltpu.make_async_copy(mbuf1, acc.at[pl.ds(0, K)], ssem1).wait()
        pltpu.make_async_copy(mbuf0, acc.at[pl.ds(0, K)], ssem0).wait()

    plsc.subcore_barrier()
    pltpu.sync_copy(acc.at[pl.ds(rbase, RPT)], outr.at[pl.ds(rbase, RPT)])

    @pl.when(s == NS - 1)
    def _wb_tail():
        pltpu.sync_copy(acc.at[pl.ds(tail, ZR)], outr.at[pl.ds(tail, ZR)])


def _layer_body(ego_a, ego_b, src_a, dst_a, val_a, src_b, dst_b, val_b,
                out_a, out_b, idx_s, idx_d, vals,
                gbuf0, gbuf1, mbuf0, mbuf1, dbuf0, dbuf1, zbuf, acc,
                gsem0, gsem1, ssem0, ssem1):
    c = lax.axis_index("c")
    s = lax.axis_index("s")

    @pl.loop(0, ZR)
    def _fill(r):
        for j in range(D // L):
            zbuf[r, pl.ds(j * L, L)] = jnp.zeros((L,), _f32)

    @pl.when(c == 0)
    def _():
        _layer_tile(ego_a, src_a, dst_a, val_a, out_a, idx_s, idx_d, vals,
                    gbuf0, gbuf1, mbuf0, mbuf1, dbuf0, dbuf1, zbuf, acc,
                    gsem0, gsem1, ssem0, ssem1, s)

    @pl.when(c == 1)
    def _():
        _layer_tile(ego_b, src_b, dst_b, val_b, out_b, idx_s, idx_d, vals,
                    gbuf0, gbuf1, mbuf0, mbuf1, dbuf0, dbuf1, zbuf, acc,
                    gsem0, gsem1, ssem0, ssem1, s)


def _score_tile(e0, e1, e2, e3, uref, iref, outr, uix, iix,
                ub0, ub1, ub2, ub3, ib0, ib1, ib2, ib3, obuf, sem, s):
    base = s * BPT
    pltpu.sync_copy(uref.at[pl.ds(base, BPT)], uix)
    pltpu.sync_copy(iref.at[pl.ds(base, BPT)], iix)

    @pl.loop(0, BPT // L)
    def _pairs(ch):
        off = ch * L
        uv = uix[pl.ds(off, L)]
        iv = iix[pl.ds(off, L)] + N_USERS
        cps = [
            pltpu.async_copy(e0.at[uv], ub0, sem),
            pltpu.async_copy(e1.at[uv], ub1, sem),
            pltpu.async_copy(e2.at[uv], ub2, sem),
            pltpu.async_copy(e3.at[uv], ub3, sem),
            pltpu.async_copy(e0.at[iv], ib0, sem),
            pltpu.async_copy(e1.at[iv], ib1, sem),
            pltpu.async_copy(e2.at[iv], ib2, sem),
            pltpu.async_copy(e3.at[iv], ib3, sem),
        ]
        for cp in cps:
            cp.wait()

        lanes = lax.iota(_i32, L)
        svec = jnp.zeros((L,), _f32)
        for e in range(L):
            acc = jnp.zeros((L,), _f32)
            for j in range(D // L):
                sl = pl.ds(j * L, L)
                us = ub0[e, sl] + ub1[e, sl] + ub2[e, sl] + ub3[e, sl]
                vs = ib0[e, sl] + ib1[e, sl] + ib2[e, sl] + ib3[e, sl]
                acc = acc + us * vs
            for m in (8, 4, 2, 1):
                acc = acc + acc.at[lanes ^ m].get(mode="promise_in_bounds")
            svec = jnp.where(lanes == e, acc, svec)
        obuf[pl.ds(off, L)] = svec * (1.0 / 16.0)

    pltpu.sync_copy(obuf, outr.at[pl.ds(base, BPT)])


def _score_body(a0, a1, a2, a3, b0, b1, b2, b3, uaref, iaref, ubref, ibref,
                out_a, out_b, uix, iix,
                ub0, ub1, ub2, ub3, ib0, ib1, ib2, ib3, obuf, sem):
    c = lax.axis_index("c")
    s = lax.axis_index("s")

    @pl.when(c == 0)
    def _():
        _score_tile(a0, a1, a2, a3, uaref, iaref, out_a, uix, iix,
                    ub0, ub1, ub2, ub3, ib0, ib1, ib2, ib3, obuf, sem, s)

    @pl.when(c == 1)
    def _():
        _score_tile(b0, b1, b2, b3, ubref, ibref, out_b, uix, iix,
                    ub0, ub1, ub2, ub3, ib0, ib1, ib2, ib3, obuf, sem, s)


_MESH = plsc.VectorSubcoreMesh(core_axis_name="c", subcore_axis_name="s")

_layer_call = pl.kernel(
    _layer_body,
    out_type=[jax.ShapeDtypeStruct((N_NODES, D), _f32)] * 2,
    mesh=_MESH,
    scratch_types=[
        pltpu.VMEM((EBLK,), _i32),
        pltpu.VMEM((EBLK,), _i32),
        pltpu.VMEM((EBLK,), _f32),
        pltpu.VMEM((K, D), _f32),
        pltpu.VMEM((K, D), _f32),
        pltpu.VMEM((K, D), _f32),
        pltpu.VMEM((K, D), _f32),
        pltpu.VMEM((K,), _i32),
        pltpu.VMEM((K,), _i32),
        pltpu.VMEM((ZR, D), _f32),
        pltpu.VMEM_SHARED((N_NODES, D), _f32),
        pltpu.SemaphoreType.DMA,
        pltpu.SemaphoreType.DMA,
        pltpu.SemaphoreType.DMA,
        pltpu.SemaphoreType.DMA,
    ],
)

_score_call = pl.kernel(
    _score_body,
    out_type=[jax.ShapeDtypeStruct((B,), _f32)] * 2,
    mesh=_MESH,
    scratch_types=[
        pltpu.VMEM((BPT,), _i32),
        pltpu.VMEM((BPT,), _i32),
        pltpu.VMEM((L, D), _f32),
        pltpu.VMEM((L, D), _f32),
        pltpu.VMEM((L, D), _f32),
        pltpu.VMEM((L, D), _f32),
        pltpu.VMEM((L, D), _f32),
        pltpu.VMEM((L, D), _f32),
        pltpu.VMEM((L, D), _f32),
        pltpu.VMEM((L, D), _f32),
        pltpu.VMEM((BPT,), _f32),
        pltpu.SemaphoreType.DMA,
    ],
)


def kernel(uA, iA, uB, iB, adj_a_idx, adj_a_val, adj_b_idx, adj_b_val,
           ua_idx, ia_idx, ub_idx, ib_idx):
    ego_a = jnp.concatenate([uA, iA], axis=0)
    ego_b = jnp.concatenate([uB, iB], axis=0)
    src_a, dst_a = adj_a_idx[1], adj_a_idx[0]
    src_b, dst_b = adj_b_idx[1], adj_b_idx[0]

    a1, b1 = _layer_call(ego_a, ego_b, src_a, dst_a, adj_a_val,
                         src_b, dst_b, adj_b_val)
    a2, b2 = _layer_call(a1, b1, src_a, dst_a, adj_a_val,
                         src_b, dst_b, adj_b_val)
    a3, b3 = _layer_call(a2, b2, src_a, dst_a, adj_a_val,
                         src_b, dst_b, adj_b_val)

    sa, sb = _score_call(ego_a, a1, a2, a3, ego_b, b1, b2, b3,
                         ua_idx, ia_idx, ub_idx, ib_idx)
    return (sa, sb)


# speedup vs baseline: 7.3323x; 1.4509x over previous
import jax
import jax.numpy as jnp
from jax import lax
from jax.experimental import pallas as pl
from jax.experimental.pallas import tpu as pltpu, tpu_sc as plsc

N_USERS = 5000
N_NODES = 10000
E = 320000
D = 128
B = 4096

NC, NS, L = 2, 16, 16
EPT = E // NS
EBLK = 4000
K = 32
SUB = EBLK // K
NBO = EPT // EBLK
DEPTH = 3
RPT = 624
ZR = 16
BPT = B // NS

_f32 = jnp.float32
_i32 = jnp.int32


def _scale_rows(gbuf, mbuf, vals, off):
    @pl.loop(0, K // L)
    def _grp(g):
        vv = vals[pl.ds(off + g * L, L)]
        for e in range(L):
            splat = jnp.full((L,), vv[e], _f32)
            r = g * L + e
            for j in range(D // L):
                sl = pl.ds(j * L, L)
                mbuf[r, sl] = gbuf[r, sl] * splat


def _sub_block(ego, acc, idx_s, idx_d, vals, gbuf, mbuf, dbuf, gsem, ssem, sb):
    off = sb * K
    pltpu.make_async_copy(ego.at[pl.ds(0, K)], gbuf, gsem).wait()

    @pl.when(sb >= DEPTH)
    def _():
        pltpu.make_async_copy(mbuf, acc.at[pl.ds(0, K)], ssem).wait()

    _scale_rows(gbuf, mbuf, vals, off)

    @pl.when(sb + DEPTH < SUB)
    def _():
        pltpu.async_copy(ego.at[idx_s.at[pl.ds((sb + DEPTH) * K, K)]],
                         gbuf, gsem)

    @pl.loop(0, K // L)
    def _dst(g):
        dbuf[pl.ds(g * L, L)] = idx_d[pl.ds(off + g * L, L)]
    pltpu.async_copy(mbuf, acc.at[dbuf], ssem, add=True)


def _layer_tile(ego, srcr, dstr, valr, outr, idx_s, idx_d, vals,
                gbuf0, gbuf1, gbuf2, mbuf0, mbuf1, mbuf2,
                dbuf0, dbuf1, dbuf2, zbuf, acc,
                gsem0, gsem1, gsem2, ssem0, ssem1, ssem2, s):
    base = s * EPT
    rbase = s * RPT
    tail = NS * RPT

    @pl.loop(0, RPT // ZR)
    def _zero(k):
        pltpu.sync_copy(zbuf, acc.at[pl.ds(rbase + k * ZR, ZR)])

    @pl.when(s == NS - 1)
    def _zero_tail():
        pltpu.sync_copy(zbuf, acc.at[pl.ds(tail, ZR)])

    plsc.subcore_barrier()

    rings = ((gbuf0, mbuf0, dbuf0, gsem0, ssem0),
             (gbuf1, mbuf1, dbuf1, gsem1, ssem1),
             (gbuf2, mbuf2, dbuf2, gsem2, ssem2))

    @pl.loop(0, NBO)
    def _blk(b):
        ebase = base + b * EBLK
        pltpu.sync_copy(srcr.at[pl.ds(ebase, EBLK)], idx_s)
        pltpu.sync_copy(dstr.at[pl.ds(ebase, EBLK)], idx_d)
        pltpu.sync_copy(valr.at[pl.ds(ebase, EBLK)], vals)

        for t in range(DEPTH):
            pltpu.async_copy(ego.at[idx_s.at[pl.ds(t * K, K)]],
                             rings[t][0], rings[t][3])

        @pl.loop(0, SUB)
        def _sub(sb):
            for t in range(DEPTH):
                gbuf, mbuf, dbuf, gsem, ssem = rings[t]

                @pl.when(sb % DEPTH == t)
                def _(gbuf=gbuf, mbuf=mbuf, dbuf=dbuf, gsem=gsem, ssem=ssem):
                    _sub_block(ego, acc, idx_s, idx_d, vals,
                               gbuf, mbuf, dbuf, gsem, ssem, sb)

        for t in range(DEPTH):
            pltpu.make_async_copy(rings[t][1], acc.at[pl.ds(0, K)],
                                  rings[t][4]).wait()

    plsc.subcore_barrier()
    pltpu.sync_copy(acc.at[pl.ds(rbase, RPT)], outr.at[pl.ds(rbase, RPT)])

    @pl.when(s == NS - 1)
    def _wb_tail():
        pltpu.sync_copy(acc.at[pl.ds(tail, ZR)], outr.at[pl.ds(tail, ZR)])


def _layer_body(ego_a, ego_b, src_a, dst_a, val_a, src_b, dst_b, val_b,
                out_a, out_b, idx_s, idx_d, vals,
                gbuf0, gbuf1, gbuf2, mbuf0, mbuf1, mbuf2,
                dbuf0, dbuf1, dbuf2, zbuf, acc,
                gsem0, gsem1, gsem2, ssem0, ssem1, ssem2):
    c = lax.axis_index("c")
    s = lax.axis_index("s")

    @pl.loop(0, ZR)
    def _fill(r):
        for j in range(D // L):
            zbuf[r, pl.ds(j * L, L)] = jnp.zeros((L,), _f32)

    @pl.when(c == 0)
    def _():
        _layer_tile(ego_a, src_a, dst_a, val_a, out_a, idx_s, idx_d, vals,
                    gbuf0, gbuf1, gbuf2, mbuf0, mbuf1, mbuf2,
                    dbuf0, dbuf1, dbuf2, zbuf, acc,
                    gsem0, gsem1, gsem2, ssem0, ssem1, ssem2, s)

    @pl.when(c == 1)
    def _():
        _layer_tile(ego_b, src_b, dst_b, val_b, out_b, idx_s, idx_d, vals,
                    gbuf0, gbuf1, gbuf2, mbuf0, mbuf1, mbuf2,
                    dbuf0, dbuf1, dbuf2, zbuf, acc,
                    gsem0, gsem1, gsem2, ssem0, ssem1, ssem2, s)


def _score_tile(e0, e1, e2, e3, uref, iref, outr, uix, iix,
                ub0, ub1, ub2, ub3, ib0, ib1, ib2, ib3, obuf, sem, s):
    base = s * BPT
    pltpu.sync_copy(uref.at[pl.ds(base, BPT)], uix)
    pltpu.sync_copy(iref.at[pl.ds(base, BPT)], iix)

    @pl.loop(0, BPT // L)
    def _pairs(ch):
        off = ch * L
        uv = uix[pl.ds(off, L)]
        iv = iix[pl.ds(off, L)] + N_USERS
        cps = [
            pltpu.async_copy(e0.at[uv], ub0, sem),
            pltpu.async_copy(e1.at[uv], ub1, sem),
            pltpu.async_copy(e2.at[uv], ub2, sem),
            pltpu.async_copy(e3.at[uv], ub3, sem),
            pltpu.async_copy(e0.at[iv], ib0, sem),
            pltpu.async_copy(e1.at[iv], ib1, sem),
            pltpu.async_copy(e2.at[iv], ib2, sem),
            pltpu.async_copy(e3.at[iv], ib3, sem),
        ]
        for cp in cps:
            cp.wait()

        lanes = lax.iota(_i32, L)
        svec = jnp.zeros((L,), _f32)
        for e in range(L):
            acc = jnp.zeros((L,), _f32)
            for j in range(D // L):
                sl = pl.ds(j * L, L)
                us = ub0[e, sl] + ub1[e, sl] + ub2[e, sl] + ub3[e, sl]
                vs = ib0[e, sl] + ib1[e, sl] + ib2[e, sl] + ib3[e, sl]
                acc = acc + us * vs
            for m in (8, 4, 2, 1):
                acc = acc + acc.at[lanes ^ m].get(mode="promise_in_bounds")
            svec = jnp.where(lanes == e, acc, svec)
        obuf[pl.ds(off, L)] = svec * (1.0 / 16.0)

    pltpu.sync_copy(obuf, outr.at[pl.ds(base, BPT)])


def _score_body(a0, a1, a2, a3, b0, b1, b2, b3, uaref, iaref, ubref, ibref,
                out_a, out_b, uix, iix,
                ub0, ub1, ub2, ub3, ib0, ib1, ib2, ib3, obuf, sem):
    c = lax.axis_index("c")
    s = lax.axis_index("s")

    @pl.when(c == 0)
    def _():
        _score_tile(a0, a1, a2, a3, uaref, iaref, out_a, uix, iix,
                    ub0, ub1, ub2, ub3, ib0, ib1, ib2, ib3, obuf, sem, s)

    @pl.when(c == 1)
    def _():
        _score_tile(b0, b1, b2, b3, ubref, ibref, out_b, uix, iix,
                    ub0, ub1, ub2, ub3, ib0, ib1, ib2, ib3, obuf, sem, s)


_MESH = plsc.VectorSubcoreMesh(core_axis_name="c", subcore_axis_name="s")

_layer_call = pl.kernel(
    _layer_body,
    out_type=[jax.ShapeDtypeStruct((N_NODES, D), _f32)] * 2,
    mesh=_MESH,
    scratch_types=[
        pltpu.VMEM((EBLK,), _i32),
        pltpu.VMEM((EBLK,), _i32),
        pltpu.VMEM((EBLK,), _f32),
        pltpu.VMEM((K, D), _f32),
        pltpu.VMEM((K, D), _f32),
        pltpu.VMEM((K, D), _f32),
        pltpu.VMEM((K, D), _f32),
        pltpu.VMEM((K, D), _f32),
        pltpu.VMEM((K, D), _f32),
        pltpu.VMEM((K,), _i32),
        pltpu.VMEM((K,), _i32),
        pltpu.VMEM((K,), _i32),
        pltpu.VMEM((ZR, D), _f32),
        pltpu.VMEM_SHARED((N_NODES, D), _f32),
        pltpu.SemaphoreType.DMA,
        pltpu.SemaphoreType.DMA,
        pltpu.SemaphoreType.DMA,
        pltpu.SemaphoreType.DMA,
        pltpu.SemaphoreType.DMA,
        pltpu.SemaphoreType.DMA,
    ],
)

_score_call = pl.kernel(
    _score_body,
    out_type=[jax.ShapeDtypeStruct((B,), _f32)] * 2,
    mesh=_MESH,
    scratch_types=[
        pltpu.VMEM((BPT,), _i32),
        pltpu.VMEM((BPT,), _i32),
        pltpu.VMEM((L, D), _f32),
        pltpu.VMEM((L, D), _f32),
        pltpu.VMEM((L, D), _f32),
        pltpu.VMEM((L, D), _f32),
        pltpu.VMEM((L, D), _f32),
        pltpu.VMEM((L, D), _f32),
        pltpu.VMEM((L, D), _f32),
        pltpu.VMEM((L, D), _f32),
        pltpu.VMEM((BPT,), _f32),
        pltpu.SemaphoreType.DMA,
    ],
)


def kernel(uA, iA, uB, iB, adj_a_idx, adj_a_val, adj_b_idx, adj_b_val,
           ua_idx, ia_idx, ub_idx, ib_idx):
    ego_a = jnp.concatenate([uA, iA], axis=0)
    ego_b = jnp.concatenate([uB, iB], axis=0)
    src_a, dst_a = adj_a_idx[1], adj_a_idx[0]
    src_b, dst_b = adj_b_idx[1], adj_b_idx[0]

    a1, b1 = _layer_call(ego_a, ego_b, src_a, dst_a, adj_a_val,
                         src_b, dst_b, adj_b_val)
    a2, b2 = _layer_call(a1, b1, src_a, dst_a, adj_a_val,
                         src_b, dst_b, adj_b_val)
    a3, b3 = _layer_call(a2, b2, src_a, dst_a, adj_a_val,
                         src_b, dst_b, adj_b_val)

    sa, sb = _score_call(ego_a, a1, a2, a3, ego_b, b1, b2, b3,
                         ua_idx, ia_idx, ub_idx, ib_idx)
    return (sa, sb)


# speedup vs baseline: 7.4249x; 1.0126x over previous
import jax
import jax.numpy as jnp
from jax import lax
from jax.experimental import pallas as pl
from jax.experimental.pallas import tpu as pltpu, tpu_sc as plsc

N_USERS = 5000
N_NODES = 10000
E = 320000
D = 128
B = 4096

NC, NS, L = 2, 16, 16
EPT = E // NS
EBLK = 4000
K = 32
SUB = EBLK // K
NBO = EPT // EBLK
DEPTH = 3
RPT = 624
TAIL = NS * RPT
ZR = 16
BPT = B // NS

_f32 = jnp.float32
_i32 = jnp.int32


def _scale_rows(gbuf, mbuf, vals, off):
    @pl.loop(0, K // L)
    def _grp(g):
        vv = vals[pl.ds(off + g * L, L)]
        for e in range(L):
            splat = jnp.full((L,), vv[e], _f32)
            r = g * L + e
            for j in range(D // L):
                sl = pl.ds(j * L, L)
                mbuf[r, sl] = gbuf[r, sl] * splat


def _gather_sel(egos, lyr, idx_slice, gbuf, gsem):
    for i, ego in enumerate(egos):
        @pl.when(lyr == i)
        def _(ego=ego):
            pltpu.async_copy(ego.at[idx_slice], gbuf, gsem)


def _sub_block(egos, lyr, acc, idx_s, idx_d, vals, gbuf, mbuf, dbuf,
               gsem, ssem, sb):
    off = sb * K
    pltpu.make_async_copy(egos[0].at[pl.ds(0, K)], gbuf, gsem).wait()

    @pl.when(sb >= DEPTH)
    def _():
        pltpu.make_async_copy(mbuf, acc.at[pl.ds(0, K)], ssem).wait()

    _scale_rows(gbuf, mbuf, vals, off)

    @pl.when(sb + DEPTH < SUB)
    def _():
        _gather_sel(egos, lyr, idx_s.at[pl.ds((sb + DEPTH) * K, K)],
                    gbuf, gsem)

    @pl.loop(0, K // L)
    def _dst(g):
        dbuf[pl.ds(g * L, L)] = idx_d[pl.ds(off + g * L, L)]
    pltpu.async_copy(mbuf, acc.at[dbuf], ssem, add=True)


def _edges_pass(egos, lyr, srcr, dstr, valr, idx_s, idx_d, vals, rings, acc,
                base):
    @pl.loop(0, NBO)
    def _blk(b):
        ebase = base + b * EBLK
        pltpu.sync_copy(srcr.at[pl.ds(ebase, EBLK)], idx_s)
        pltpu.sync_copy(dstr.at[pl.ds(ebase, EBLK)], idx_d)
        pltpu.sync_copy(valr.at[pl.ds(ebase, EBLK)], vals)

        for t in range(DEPTH):
            _gather_sel(egos, lyr, idx_s.at[pl.ds(t * K, K)],
                        rings[t][0], rings[t][3])

        @pl.loop(0, SUB)
        def _sub(sb):
            for t in range(DEPTH):
                gbuf, mbuf, dbuf, gsem, ssem = rings[t]

                @pl.when(sb % DEPTH == t)
                def _(gbuf=gbuf, mbuf=mbuf, dbuf=dbuf, gsem=gsem, ssem=ssem):
                    _sub_block(egos, lyr, acc, idx_s, idx_d, vals,
                               gbuf, mbuf, dbuf, gsem, ssem, sb)

        for t in range(DEPTH):
            pltpu.make_async_copy(rings[t][1], acc.at[pl.ds(0, K)],
                                  rings[t][4]).wait()


def _score_part(e0, e1, e2, e3, uref, iref, outr, uix, iix,
                gbuf0, gbuf1, gbuf2, mbuf0, obuf, sem, s):
    base = s * BPT
    pltpu.sync_copy(uref.at[pl.ds(base, BPT)], uix)
    pltpu.sync_copy(iref.at[pl.ds(base, BPT)], iix)

    @pl.loop(0, BPT // L)
    def _pairs(ch):
        off = ch * L
        uv = uix[pl.ds(off, L)]
        iv = iix[pl.ds(off, L)] + N_USERS
        cps = [
            pltpu.async_copy(e0.at[uv], gbuf0.at[pl.ds(0, L)], sem),
            pltpu.async_copy(e1.at[uv], gbuf0.at[pl.ds(L, L)], sem),
            pltpu.async_copy(e2.at[uv], gbuf1.at[pl.ds(0, L)], sem),
            pltpu.async_copy(e3.at[uv], gbuf1.at[pl.ds(L, L)], sem),
            pltpu.async_copy(e0.at[iv], gbuf2.at[pl.ds(0, L)], sem),
            pltpu.async_copy(e1.at[iv], gbuf2.at[pl.ds(L, L)], sem),
            pltpu.async_copy(e2.at[iv], mbuf0.at[pl.ds(0, L)], sem),
            pltpu.async_copy(e3.at[iv], mbuf0.at[pl.ds(L, L)], sem),
        ]
        for cp in cps:
            cp.wait()

        lanes = lax.iota(_i32, L)
        svec = jnp.zeros((L,), _f32)
        for e in range(L):
            acc = jnp.zeros((L,), _f32)
            for j in range(D // L):
                sl = pl.ds(j * L, L)
                us = (gbuf0[e, sl] + gbuf0[L + e, sl]
                      + gbuf1[e, sl] + gbuf1[L + e, sl])
                vs = (gbuf2[e, sl] + gbuf2[L + e, sl]
                      + mbuf0[e, sl] + mbuf0[L + e, sl])
                acc = acc + us * vs
            for m in (8, 4, 2, 1):
                acc = acc + acc.at[lanes ^ m].get(mode="promise_in_bounds")
            svec = jnp.where(lanes == e, acc, svec)
        obuf[pl.ds(off, L)] = svec * (1.0 / 16.0)

    pltpu.sync_copy(obuf, outr.at[pl.ds(base, BPT)])


def _domain_tile(e0, srcr, dstr, valr, e1, e2, e3,
                 idx_s, idx_d, vals, rings, zbuf, acc, s):
    base = s * EPT
    rbase = s * RPT

    def zero_own():
        @pl.loop(0, RPT // ZR)
        def _zero(k):
            pltpu.sync_copy(zbuf, acc.at[pl.ds(rbase + k * ZR, ZR)])

        @pl.when(s == NS - 1)
        def _zero_tail():
            pltpu.sync_copy(zbuf, acc.at[pl.ds(TAIL, ZR)])

    def writeback(eout):
        pltpu.sync_copy(acc.at[pl.ds(rbase, RPT)], eout.at[pl.ds(rbase, RPT)])

        @pl.when(s == NS - 1)
        def _wb_tail():
            pltpu.sync_copy(acc.at[pl.ds(TAIL, ZR)], eout.at[pl.ds(TAIL, ZR)])

    zero_own()
    plsc.subcore_barrier()

    egos = (e0, e1, e2)
    eouts = (e1, e2, e3)

    @pl.loop(0, 3)
    def _layer(lyr):
        _edges_pass(egos, lyr, srcr, dstr, valr, idx_s, idx_d, vals, rings,
                    acc, base)
        plsc.subcore_barrier()
        for i, eout in enumerate(eouts):
            @pl.when(lyr == i)
            def _(eout=eout):
                writeback(eout)

        @pl.when(lyr < 2)
        def _():
            zero_own()
        plsc.subcore_barrier()


def _layers_body(ego_a, ego_b, src_a, dst_a, val_a, src_b, dst_b, val_b,
                 a1, a2, a3, b1, b2, b3,
                 idx_s, idx_d, vals,
                 gbuf0, gbuf1, gbuf2, mbuf0, mbuf1, mbuf2,
                 dbuf0, dbuf1, dbuf2, zbuf, acc,
                 gsem0, gsem1, gsem2, ssem0, ssem1, ssem2):
    c = lax.axis_index("c")
    s = lax.axis_index("s")

    @pl.loop(0, ZR)
    def _fill(r):
        for j in range(D // L):
            zbuf[r, pl.ds(j * L, L)] = jnp.zeros((L,), _f32)

    rings = ((gbuf0, mbuf0, dbuf0, gsem0, ssem0),
             (gbuf1, mbuf1, dbuf1, gsem1, ssem1),
             (gbuf2, mbuf2, dbuf2, gsem2, ssem2))

    @pl.when(c == 0)
    def _():
        _domain_tile(ego_a, src_a, dst_a, val_a, a1, a2, a3,
                     idx_s, idx_d, vals, rings, zbuf, acc, s)

    @pl.when(c == 1)
    def _():
        _domain_tile(ego_b, src_b, dst_b, val_b, b1, b2, b3,
                     idx_s, idx_d, vals, rings, zbuf, acc, s)


_MESH = plsc.VectorSubcoreMesh(core_axis_name="c", subcore_axis_name="s")

_layers_call = pl.kernel(
    _layers_body,
    out_type=[jax.ShapeDtypeStruct((N_NODES, D), _f32)] * 6,
    mesh=_MESH,
    scratch_types=[
        pltpu.VMEM((EBLK,), _i32),
        pltpu.VMEM((EBLK,), _i32),
        pltpu.VMEM((EBLK,), _f32),
        pltpu.VMEM((K, D), _f32),
        pltpu.VMEM((K, D), _f32),
        pltpu.VMEM((K, D), _f32),
        pltpu.VMEM((K, D), _f32),
        pltpu.VMEM((K, D), _f32),
        pltpu.VMEM((K, D), _f32),
        pltpu.VMEM((K,), _i32),
        pltpu.VMEM((K,), _i32),
        pltpu.VMEM((K,), _i32),
        pltpu.VMEM((ZR, D), _f32),
        pltpu.VMEM_SHARED((N_NODES, D), _f32),
        pltpu.SemaphoreType.DMA,
        pltpu.SemaphoreType.DMA,
        pltpu.SemaphoreType.DMA,
        pltpu.SemaphoreType.DMA,
        pltpu.SemaphoreType.DMA,
        pltpu.SemaphoreType.DMA,
    ],
)


def _score_body(a0, a1, a2, a3, b0, b1, b2, b3, uaref, iaref, ubref, ibref,
                sa, sb, uix, iix, gbuf0, gbuf1, gbuf2, mbuf0, obuf, sem):
    c = lax.axis_index("c")
    s = lax.axis_index("s")

    @pl.when(c == 0)
    def _():
        _score_part(a0, a1, a2, a3, uaref, iaref, sa, uix, iix,
                    gbuf0, gbuf1, gbuf2, mbuf0, obuf, sem, s)

    @pl.when(c == 1)
    def _():
        _score_part(b0, b1, b2, b3, ubref, ibref, sb, uix, iix,
                    gbuf0, gbuf1, gbuf2, mbuf0, obuf, sem, s)


_score_call = pl.kernel(
    _score_body,
    out_type=[jax.ShapeDtypeStruct((B,), _f32)] * 2,
    mesh=_MESH,
    scratch_types=[
        pltpu.VMEM((BPT,), _i32),
        pltpu.VMEM((BPT,), _i32),
        pltpu.VMEM((K, D), _f32),
        pltpu.VMEM((K, D), _f32),
        pltpu.VMEM((K, D), _f32),
        pltpu.VMEM((K, D), _f32),
        pltpu.VMEM((BPT,), _f32),
        pltpu.SemaphoreType.DMA,
    ],
)


def kernel(uA, iA, uB, iB, adj_a_idx, adj_a_val, adj_b_idx, adj_b_val,
           ua_idx, ia_idx, ub_idx, ib_idx):
    ego_a = jnp.concatenate([uA, iA], axis=0)
    ego_b = jnp.concatenate([uB, iB], axis=0)
    src_a, dst_a = adj_a_idx[1], adj_a_idx[0]
    src_b, dst_b = adj_b_idx[1], adj_b_idx[0]

    a1, a2, a3, b1, b2, b3 = _layers_call(ego_a, ego_b,
                                          src_a, dst_a, adj_a_val,
                                          src_b, dst_b, adj_b_val)
    sa, sb = _score_call(ego_a, a1, a2, a3, ego_b, b1, b2, b3,
                         ua_idx, ia_idx, ub_idx, ib_idx)
    return (sa, sb)


# speedup vs baseline: 8.1535x; 1.0981x over previous
import jax
import jax.numpy as jnp
from jax import lax
from jax.experimental import pallas as pl
from jax.experimental.pallas import tpu as pltpu, tpu_sc as plsc

N_USERS = 5000
N_NODES = 10000
E = 320000
D = 128
B = 4096

NC, NS, L = 2, 16, 16
EPT = E // NS
EBLK = 4000
K = 32
SUB = EBLK // K
NBO = EPT // EBLK
DEPTH = 4
RPT = 624
TAIL = NS * RPT
ZR = 16
BPT = B // NS

_f32 = jnp.float32
_i32 = jnp.int32


def _scale_rows(gbuf, mbuf, vals, off):
    @pl.loop(0, K // L)
    def _grp(g):
        vv = vals[pl.ds(off + g * L, L)]
        for e in range(L):
            splat = jnp.full((L,), vv[e], _f32)
            r = g * L + e
            for j in range(D // L):
                sl = pl.ds(j * L, L)
                mbuf[r, sl] = gbuf[r, sl] * splat


def _gather_sel(egos, lyr, idx_slice, gbuf, gsem):
    for i, ego in enumerate(egos):
        @pl.when(lyr == i)
        def _(ego=ego):
            pltpu.async_copy(ego.at[idx_slice], gbuf, gsem)


def _sub_block(egos, lyr, acc, idx_s, idx_d, vals, gbuf, mbuf, dbuf,
               gsem, ssem, sb):
    off = sb * K
    pltpu.make_async_copy(egos[0].at[pl.ds(0, K)], gbuf, gsem).wait()

    @pl.when(sb >= DEPTH)
    def _():
        pltpu.make_async_copy(mbuf, acc.at[pl.ds(0, K)], ssem).wait()

    _scale_rows(gbuf, mbuf, vals, off)

    @pl.when(sb + DEPTH < SUB)
    def _():
        _gather_sel(egos, lyr, idx_s.at[pl.ds((sb + DEPTH) * K, K)],
                    gbuf, gsem)

    @pl.loop(0, K // L)
    def _dst(g):
        dbuf[pl.ds(g * L, L)] = idx_d[pl.ds(off + g * L, L)]
    pltpu.async_copy(mbuf, acc.at[dbuf], ssem, add=True)


def _edges_pass(egos, lyr, srcr, dstr, valr, idx_s, idx_d, vals, rings, acc,
                base):
    @pl.loop(0, NBO)
    def _blk(b):
        ebase = base + b * EBLK
        pltpu.sync_copy(srcr.at[pl.ds(ebase, EBLK)], idx_s)
        pltpu.sync_copy(dstr.at[pl.ds(ebase, EBLK)], idx_d)
        pltpu.sync_copy(valr.at[pl.ds(ebase, EBLK)], vals)

        for t in range(DEPTH):
            _gather_sel(egos, lyr, idx_s.at[pl.ds(t * K, K)],
                        rings[t][0], rings[t][3])

        @pl.loop(0, SUB)
        def _sub(sb):
            for t in range(DEPTH):
                gbuf, mbuf, dbuf, gsem, ssem = rings[t]

                @pl.when(sb % DEPTH == t)
                def _(gbuf=gbuf, mbuf=mbuf, dbuf=dbuf, gsem=gsem, ssem=ssem):
                    _sub_block(egos, lyr, acc, idx_s, idx_d, vals,
                               gbuf, mbuf, dbuf, gsem, ssem, sb)

        for t in range(DEPTH):
            pltpu.make_async_copy(rings[t][1], acc.at[pl.ds(0, K)],
                                  rings[t][4]).wait()


def _score_issue(e0, e1, e2, e3, uix, iix, ch, bset, sem):
    off = ch * L
    uv = uix[pl.ds(off, L)]
    iv = iix[pl.ds(off, L)] + N_USERS
    b0, b1, b2, b3 = bset
    pltpu.async_copy(e0.at[uv], b0.at[pl.ds(0, L)], sem)
    pltpu.async_copy(e1.at[uv], b0.at[pl.ds(L, L)], sem)
    pltpu.async_copy(e2.at[uv], b1.at[pl.ds(0, L)], sem)
    pltpu.async_copy(e3.at[uv], b1.at[pl.ds(L, L)], sem)
    pltpu.async_copy(e0.at[iv], b2.at[pl.ds(0, L)], sem)
    pltpu.async_copy(e1.at[iv], b2.at[pl.ds(L, L)], sem)
    pltpu.async_copy(e2.at[iv], b3.at[pl.ds(0, L)], sem)
    pltpu.async_copy(e3.at[iv], b3.at[pl.ds(L, L)], sem)


def _score_chunk(e0, ch, bset, obuf, sem):
    b0, b1, b2, b3 = bset
    for half in (0, L):
        for b in (b0, b1, b2, b3):
            pltpu.make_async_copy(e0.at[pl.ds(0, L)],
                                  b.at[pl.ds(half, L)], sem).wait()

    lanes = lax.iota(_i32, L)
    svec = jnp.zeros((L,), _f32)
    for e in range(L):
        acc = jnp.zeros((L,), _f32)
        for j in range(D // L):
            sl = pl.ds(j * L, L)
            us = (b0[e, sl] + b0[L + e, sl] + b1[e, sl] + b1[L + e, sl])
            vs = (b2[e, sl] + b2[L + e, sl] + b3[e, sl] + b3[L + e, sl])
            acc = acc + us * vs
        for m in (8, 4, 2, 1):
            acc = acc + acc.at[lanes ^ m].get(mode="promise_in_bounds")
        svec = jnp.where(lanes == e, acc, svec)
    obuf[pl.ds(ch * L, L)] = svec * (1.0 / 16.0)


def _score_part(e0, e1, e2, e3, uref, iref, outr, uix, iix,
                bset0, bset1, obuf, sem0, sem1, s):
    base = s * BPT
    pltpu.sync_copy(uref.at[pl.ds(base, BPT)], uix)
    pltpu.sync_copy(iref.at[pl.ds(base, BPT)], iix)

    _score_issue(e0, e1, e2, e3, uix, iix, 0, bset0, sem0)

    @pl.loop(0, BPT // L)
    def _pairs(ch):
        for par, (bset, sem, obset, osem) in enumerate(
                ((bset0, sem0, bset1, sem1), (bset1, sem1, bset0, sem0))):
            @pl.when(ch % 2 == par)
            def _(bset=bset, sem=sem, obset=obset, osem=osem):
                @pl.when(ch + 1 < BPT // L)
                def _():
                    _score_issue(e0, e1, e2, e3, uix, iix, ch + 1,
                                 obset, osem)
                _score_chunk(e0, ch, bset, obuf, sem)

    pltpu.sync_copy(obuf, outr.at[pl.ds(base, BPT)])


def _domain_tile(e0, srcr, dstr, valr, e1, e2, e3,
                 idx_s, idx_d, vals, rings, zbuf, acc, s):
    base = s * EPT
    rbase = s * RPT

    def zero_own():
        @pl.loop(0, RPT // ZR)
        def _zero(k):
            pltpu.sync_copy(zbuf, acc.at[pl.ds(rbase + k * ZR, ZR)])

        @pl.when(s == NS - 1)
        def _zero_tail():
            pltpu.sync_copy(zbuf, acc.at[pl.ds(TAIL, ZR)])

    def writeback(eout):
        pltpu.sync_copy(acc.at[pl.ds(rbase, RPT)], eout.at[pl.ds(rbase, RPT)])

        @pl.when(s == NS - 1)
        def _wb_tail():
            pltpu.sync_copy(acc.at[pl.ds(TAIL, ZR)], eout.at[pl.ds(TAIL, ZR)])

    zero_own()
    plsc.subcore_barrier()

    egos = (e0, e1, e2)
    eouts = (e1, e2, e3)

    @pl.loop(0, 3)
    def _layer(lyr):
        _edges_pass(egos, lyr, srcr, dstr, valr, idx_s, idx_d, vals, rings,
                    acc, base)
        plsc.subcore_barrier()
        for i, eout in enumerate(eouts):
            @pl.when(lyr == i)
            def _(eout=eout):
                writeback(eout)

        @pl.when(lyr < 2)
        def _():
            zero_own()
        plsc.subcore_barrier()


def _layers_body(ego_a, ego_b, src_a, dst_a, val_a, src_b, dst_b, val_b,
                 a1, a2, a3, b1, b2, b3,
                 idx_s, idx_d, vals,
                 gbuf0, gbuf1, gbuf2, gbuf3, mbuf0, mbuf1, mbuf2, mbuf3,
                 dbuf0, dbuf1, dbuf2, dbuf3, zbuf, acc,
                 gsem0, gsem1, gsem2, gsem3, ssem0, ssem1, ssem2, ssem3):
    c = lax.axis_index("c")
    s = lax.axis_index("s")

    @pl.loop(0, ZR)
    def _fill(r):
        for j in range(D // L):
            zbuf[r, pl.ds(j * L, L)] = jnp.zeros((L,), _f32)

    rings = ((gbuf0, mbuf0, dbuf0, gsem0, ssem0),
             (gbuf1, mbuf1, dbuf1, gsem1, ssem1),
             (gbuf2, mbuf2, dbuf2, gsem2, ssem2),
             (gbuf3, mbuf3, dbuf3, gsem3, ssem3))

    @pl.when(c == 0)
    def _():
        _domain_tile(ego_a, src_a, dst_a, val_a, a1, a2, a3,
                     idx_s, idx_d, vals, rings, zbuf, acc, s)

    @pl.when(c == 1)
    def _():
        _domain_tile(ego_b, src_b, dst_b, val_b, b1, b2, b3,
                     idx_s, idx_d, vals, rings, zbuf, acc, s)


_MESH = plsc.VectorSubcoreMesh(core_axis_name="c", subcore_axis_name="s")

_layers_call = pl.kernel(
    _layers_body,
    out_type=[jax.ShapeDtypeStruct((N_NODES, D), _f32)] * 6,
    mesh=_MESH,
    scratch_types=[
        pltpu.VMEM((EBLK,), _i32),
        pltpu.VMEM((EBLK,), _i32),
        pltpu.VMEM((EBLK,), _f32),
        pltpu.VMEM((K, D), _f32),
        pltpu.VMEM((K, D), _f32),
        pltpu.VMEM((K, D), _f32),
        pltpu.VMEM((K, D), _f32),
        pltpu.VMEM((K, D), _f32),
        pltpu.VMEM((K, D), _f32),
        pltpu.VMEM((K, D), _f32),
        pltpu.VMEM((K, D), _f32),
        pltpu.VMEM((K,), _i32),
        pltpu.VMEM((K,), _i32),
        pltpu.VMEM((K,), _i32),
        pltpu.VMEM((K,), _i32),
        pltpu.VMEM((ZR, D), _f32),
        pltpu.VMEM_SHARED((N_NODES, D), _f32),
    ] + [pltpu.SemaphoreType.DMA] * 8,
)


def _score_body(a0, a1, a2, a3, b0, b1, b2, b3, uaref, iaref, ubref, ibref,
                sa, sb, uix, iix, p0, p1, p2, p3, q0, q1, q2, q3, obuf,
                sem0, sem1):
    c = lax.axis_index("c")
    s = lax.axis_index("s")
    bset0 = (p0, p1, p2, p3)
    bset1 = (q0, q1, q2, q3)

    @pl.when(c == 0)
    def _():
        _score_part(a0, a1, a2, a3, uaref, iaref, sa, uix, iix,
                    bset0, bset1, obuf, sem0, sem1, s)

    @pl.when(c == 1)
    def _():
        _score_part(b0, b1, b2, b3, ubref, ibref, sb, uix, iix,
                    bset0, bset1, obuf, sem0, sem1, s)


_score_call = pl.kernel(
    _score_body,
    out_type=[jax.ShapeDtypeStruct((B,), _f32)] * 2,
    mesh=_MESH,
    scratch_types=[
        pltpu.VMEM((BPT,), _i32),
        pltpu.VMEM((BPT,), _i32),
    ] + [pltpu.VMEM((2 * L, D), _f32)] * 8
    + [
        pltpu.VMEM((BPT,), _f32),
        pltpu.SemaphoreType.DMA,
        pltpu.SemaphoreType.DMA,
    ],
)


def kernel(uA, iA, uB, iB, adj_a_idx, adj_a_val, adj_b_idx, adj_b_val,
           ua_idx, ia_idx, ub_idx, ib_idx):
    ego_a = jnp.concatenate([uA, iA], axis=0)
    ego_b = jnp.concatenate([uB, iB], axis=0)
    src_a, dst_a = adj_a_idx[1], adj_a_idx[0]
    src_b, dst_b = adj_b_idx[1], adj_b_idx[0]

    a1, a2, a3, b1, b2, b3 = _layers_call(ego_a, ego_b,
                                          src_a, dst_a, adj_a_val,
                                          src_b, dst_b, adj_b_val)
    sa, sb = _score_call(ego_a, a1, a2, a3, ego_b, b1, b2, b3,
                         ua_idx, ia_idx, ub_idx, ib_idx)
    return (sa, sb)


# speedup vs baseline: 8.8098x; 1.0805x over previous
import jax
import jax.numpy as jnp
from jax import lax
from jax.experimental import pallas as pl
from jax.experimental.pallas import tpu as pltpu, tpu_sc as plsc

N_USERS = 5000
N_NODES = 10000
E = 320000
D = 128
B = 4096

NC, NS, L = 2, 16, 16
EPT = E // NS
EBLK = 4000
K = 32
SUB = EBLK // K
NBO = EPT // EBLK
DEPTH = 4
RPT = 624
TAIL = NS * RPT
ZR = 16
BPT = B // NS
SSETS = 4

_f32 = jnp.float32
_i32 = jnp.int32


def _scale_rows(gbuf, mbuf, vals, off):
    @pl.loop(0, K // L)
    def _grp(g):
        vv = vals[pl.ds(off + g * L, L)]
        for e in range(L):
            splat = jnp.full((L,), vv[e], _f32)
            r = g * L + e
            for j in range(D // L):
                sl = pl.ds(j * L, L)
                mbuf[r, sl] = gbuf[r, sl] * splat


def _gather_sel(egos, lyr, idx_slice, gbuf, gsem):
    for i, ego in enumerate(egos):
        @pl.when(lyr == i)
        def _(ego=ego):
            pltpu.async_copy(ego.at[idx_slice], gbuf, gsem)


def _sub_block(egos, lyr, acc, idx_s, idx_d, vals, gbuf, mbuf, dbuf,
               gsem, ssem, sb):
    off = sb * K
    pltpu.make_async_copy(egos[0].at[pl.ds(0, K)], gbuf, gsem).wait()

    @pl.when(sb >= DEPTH)
    def _():
        pltpu.make_async_copy(mbuf, acc.at[pl.ds(0, K)], ssem).wait()

    _scale_rows(gbuf, mbuf, vals, off)

    @pl.when(sb + DEPTH < SUB)
    def _():
        _gather_sel(egos, lyr, idx_s.at[pl.ds((sb + DEPTH) * K, K)],
                    gbuf, gsem)

    @pl.loop(0, K // L)
    def _dst(g):
        dbuf[pl.ds(g * L, L)] = idx_d[pl.ds(off + g * L, L)]
    pltpu.async_copy(mbuf, acc.at[dbuf], ssem, add=True)


def _edges_pass(egos, lyr, srcr, dstr, valr, idx_s, idx_d, vals, rings, acc,
                base):
    @pl.loop(0, NBO)
    def _blk(b):
        ebase = base + b * EBLK
        pltpu.sync_copy(srcr.at[pl.ds(ebase, EBLK)], idx_s)
        pltpu.sync_copy(dstr.at[pl.ds(ebase, EBLK)], idx_d)
        pltpu.sync_copy(valr.at[pl.ds(ebase, EBLK)], vals)

        for t in range(DEPTH):
            _gather_sel(egos, lyr, idx_s.at[pl.ds(t * K, K)],
                        rings[t][0], rings[t][3])

        @pl.loop(0, SUB)
        def _sub(sb):
            for t in range(DEPTH):
                gbuf, mbuf, dbuf, gsem, ssem = rings[t]

                @pl.when(sb % DEPTH == t)
                def _(gbuf=gbuf, mbuf=mbuf, dbuf=dbuf, gsem=gsem, ssem=ssem):
                    _sub_block(egos, lyr, acc, idx_s, idx_d, vals,
                               gbuf, mbuf, dbuf, gsem, ssem, sb)

        for t in range(DEPTH):
            pltpu.make_async_copy(rings[t][1], acc.at[pl.ds(0, K)],
                                  rings[t][4]).wait()


def _score_issue(e0, e1, e2, e3, uix, iix, ch, bset, sem):
    off = ch * L
    uv = uix[pl.ds(off, L)]
    iv = iix[pl.ds(off, L)] + N_USERS
    b0, b1, b2, b3 = bset
    pltpu.async_copy(e0.at[uv], b0.at[pl.ds(0, L)], sem)
    pltpu.async_copy(e1.at[uv], b0.at[pl.ds(L, L)], sem)
    pltpu.async_copy(e2.at[uv], b1.at[pl.ds(0, L)], sem)
    pltpu.async_copy(e3.at[uv], b1.at[pl.ds(L, L)], sem)
    pltpu.async_copy(e0.at[iv], b2.at[pl.ds(0, L)], sem)
    pltpu.async_copy(e1.at[iv], b2.at[pl.ds(L, L)], sem)
    pltpu.async_copy(e2.at[iv], b3.at[pl.ds(0, L)], sem)
    pltpu.async_copy(e3.at[iv], b3.at[pl.ds(L, L)], sem)


def _score_chunk(e0, ch, bset, obuf, sem):
    b0, b1, b2, b3 = bset
    for half in (0, L):
        for b in (b0, b1, b2, b3):
            pltpu.make_async_copy(e0.at[pl.ds(0, L)],
                                  b.at[pl.ds(half, L)], sem).wait()

    lanes = lax.iota(_i32, L)

    @pl.loop(0, L, init_carry=jnp.zeros((L,), _f32))
    def _pair(e, svec):
        acc = jnp.zeros((L,), _f32)
        for j in range(D // L):
            sl = pl.ds(j * L, L)
            us = (b0[e, sl] + b0[L + e, sl] + b1[e, sl] + b1[L + e, sl])
            vs = (b2[e, sl] + b2[L + e, sl] + b3[e, sl] + b3[L + e, sl])
            acc = acc + us * vs
        for m in (8, 4, 2, 1):
            acc = acc + acc.at[lanes ^ m].get(mode="promise_in_bounds")
        return jnp.where(lanes == e, acc, svec)

    obuf[pl.ds(ch * L, L)] = _pair * (1.0 / 16.0)


def _score_part(e0, e1, e2, e3, uref, iref, outr, uix, iix,
                bsets, obuf, sems, s):
    base = s * BPT
    pltpu.sync_copy(uref.at[pl.ds(base, BPT)], uix)
    pltpu.sync_copy(iref.at[pl.ds(base, BPT)], iix)

    for t in range(SSETS):
        _score_issue(e0, e1, e2, e3, uix, iix, t, bsets[t], sems[t])

    @pl.loop(0, BPT // L)
    def _pairs(ch):
        for par in range(SSETS):
            @pl.when(ch % SSETS == par)
            def _(bset=bsets[par], sem=sems[par]):
                _score_chunk(e0, ch, bset, obuf, sem)
                @pl.when(ch + SSETS < BPT // L)
                def _():
                    _score_issue(e0, e1, e2, e3, uix, iix, ch + SSETS,
                                 bset, sem)

    pltpu.sync_copy(obuf, outr.at[pl.ds(base, BPT)])


def _domain_tile(e0, srcr, dstr, valr, e1, e2, e3,
                 idx_s, idx_d, vals, rings, zbuf, acc, s):
    base = s * EPT
    rbase = s * RPT

    def zero_own():
        @pl.loop(0, RPT // ZR)
        def _zero(k):
            pltpu.sync_copy(zbuf, acc.at[pl.ds(rbase + k * ZR, ZR)])

        @pl.when(s == NS - 1)
        def _zero_tail():
            pltpu.sync_copy(zbuf, acc.at[pl.ds(TAIL, ZR)])

    def writeback(eout):
        pltpu.sync_copy(acc.at[pl.ds(rbase, RPT)], eout.at[pl.ds(rbase, RPT)])

        @pl.when(s == NS - 1)
        def _wb_tail():
            pltpu.sync_copy(acc.at[pl.ds(TAIL, ZR)], eout.at[pl.ds(TAIL, ZR)])

    zero_own()
    plsc.subcore_barrier()

    egos = (e0, e1, e2)
    eouts = (e1, e2, e3)

    @pl.loop(0, 3)
    def _layer(lyr):
        _edges_pass(egos, lyr, srcr, dstr, valr, idx_s, idx_d, vals, rings,
                    acc, base)
        plsc.subcore_barrier()
        for i, eout in enumerate(eouts):
            @pl.when(lyr == i)
            def _(eout=eout):
                writeback(eout)

        @pl.when(lyr < 2)
        def _():
            zero_own()
        plsc.subcore_barrier()


def _layers_body(ego_a, ego_b, src_a, dst_a, val_a, src_b, dst_b, val_b,
                 a1, a2, a3, b1, b2, b3,
                 idx_s, idx_d, vals, gbufs, mbufs, dbufs, zbuf, acc, gsems,
                 ssems):
    c = lax.axis_index("c")
    s = lax.axis_index("s")

    @pl.loop(0, ZR)
    def _fill(r):
        for j in range(D // L):
            zbuf[r, pl.ds(j * L, L)] = jnp.zeros((L,), _f32)

    rings = tuple(zip(gbufs, mbufs, dbufs, gsems, ssems))

    @pl.when(c == 0)
    def _():
        _domain_tile(ego_a, src_a, dst_a, val_a, a1, a2, a3,
                     idx_s, idx_d, vals, rings, zbuf, acc, s)

    @pl.when(c == 1)
    def _():
        _domain_tile(ego_b, src_b, dst_b, val_b, b1, b2, b3,
                     idx_s, idx_d, vals, rings, zbuf, acc, s)


_MESH = plsc.VectorSubcoreMesh(core_axis_name="c", subcore_axis_name="s")

_layers_call = pl.kernel(
    _layers_body,
    out_type=[jax.ShapeDtypeStruct((N_NODES, D), _f32)] * 6,
    mesh=_MESH,
    scratch_types=[
        pltpu.VMEM((EBLK,), _i32),
        pltpu.VMEM((EBLK,), _i32),
        pltpu.VMEM((EBLK,), _f32),
        [pltpu.VMEM((K, D), _f32)] * DEPTH,
        [pltpu.VMEM((K, D), _f32)] * DEPTH,
        [pltpu.VMEM((K,), _i32)] * DEPTH,
        pltpu.VMEM((ZR, D), _f32),
        pltpu.VMEM_SHARED((N_NODES, D), _f32),
        [pltpu.SemaphoreType.DMA] * DEPTH,
        [pltpu.SemaphoreType.DMA] * DEPTH,
    ],
)


def _score_body(a0, a1, a2, a3, b0, b1, b2, b3, uaref, iaref, ubref, ibref,
                sa, sb, uix, iix, bufs, obuf, sems):
    c = lax.axis_index("c")
    s = lax.axis_index("s")
    bsets = tuple(tuple(bufs[4 * t:4 * t + 4]) for t in range(SSETS))

    @pl.when(c == 0)
    def _():
        _score_part(a0, a1, a2, a3, uaref, iaref, sa, uix, iix,
                    bsets, obuf, sems, s)

    @pl.when(c == 1)
    def _():
        _score_part(b0, b1, b2, b3, ubref, ibref, sb, uix, iix,
                    bsets, obuf, sems, s)


_score_call = pl.kernel(
    _score_body,
    out_type=[jax.ShapeDtypeStruct((B,), _f32)] * 2,
    mesh=_MESH,
    scratch_types=[
        pltpu.VMEM((BPT,), _i32),
        pltpu.VMEM((BPT,), _i32),
        [pltpu.VMEM((2 * L, D), _f32)] * (4 * SSETS),
        pltpu.VMEM((BPT,), _f32),
        [pltpu.SemaphoreType.DMA] * SSETS,
    ],
)


def kernel(uA, iA, uB, iB, adj_a_idx, adj_a_val, adj_b_idx, adj_b_val,
           ua_idx, ia_idx, ub_idx, ib_idx):
    ego_a = jnp.concatenate([uA, iA], axis=0)
    ego_b = jnp.concatenate([uB, iB], axis=0)
    src_a, dst_a = adj_a_idx[1], adj_a_idx[0]
    src_b, dst_b = adj_b_idx[1], adj_b_idx[0]

    a1, a2, a3, b1, b2, b3 = _layers_call(ego_a, ego_b,
                                          src_a, dst_a, adj_a_val,
                                          src_b, dst_b, adj_b_val)
    sa, sb = _score_call(ego_a, a1, a2, a3, ego_b, b1, b2, b3,
                         ua_idx, ia_idx, ub_idx, ib_idx)
    return (sa, sb)
